# baseline (speedup 1.0000x reference)
import functools

import jax
import jax.numpy as jnp
from jax import lax
from jax.experimental import pallas as pl
from jax.experimental.pallas import tpu as pltpu
from jax.experimental.pallas import tpu_sc as plsc

VOCAB = 1000000
EMBED_DIM = 32
B = 16384
L = 50

NUM_WORKERS = 32
NTB = B // 128
TB_PER_WORKER = NTB // NUM_WORKERS
NTD = EMBED_DIM // 8

LC = 5
SLOTS_PER_TB = L // LC
NSLOTS = TB_PER_WORKER * SLOTS_PER_TB


def _sc_gather_relu(idx_t, table):
    mesh = plsc.VectorSubcoreMesh(core_axis_name="c", subcore_axis_name="s")

    @functools.partial(
        pl.kernel,
        mesh=mesh,
        out_type=jax.ShapeDtypeStruct((L, NTD, NTB, 8, 128, 1), jnp.float32),
        scratch_types=[
            pltpu.VMEM((TB_PER_WORKER, L, 128), jnp.int32),
            pltpu.VMEM((LC * 128, EMBED_DIM), jnp.float32),
            pltpu.VMEM((LC * 128, EMBED_DIM), jnp.float32),
            pltpu.VMEM((LC * 128, EMBED_DIM), jnp.float32),
            pltpu.VMEM((LC * 128, EMBED_DIM), jnp.float32),
            pltpu.SemaphoreType.DMA,
            pltpu.SemaphoreType.DMA,
            pltpu.SemaphoreType.DMA,
            pltpu.SemaphoreType.DMA,
            pltpu.SemaphoreType.DMA,
            pltpu.SemaphoreType.DMA,
            pltpu.SemaphoreType.DMA,
            pltpu.SemaphoreType.DMA,
        ],
        compiler_params=pltpu.CompilerParams(use_tc_tiling_on_sc=False),
    )
    def k(idx_hbm, table_hbm, out_hbm, idxt_v,
          ga, gb, gc, gd, gsa, gsb, gsc, gsd, osa, osb, osc, osd):
        wid = lax.axis_index("c") * 16 + lax.axis_index("s")
        tb0 = wid * TB_PER_WORKER

        gbuf = (ga, gb, gc, gd)
        gsem = (gsa, gsb, gsc, gsd)
        osem = (osa, osb, osc, osd)

        for t in range(TB_PER_WORKER):
            pltpu.sync_copy(
                idx_hbm.at[pl.ds(0, L), pl.ds((tb0 + t) * 128, 128)],
                idxt_v.at[t],
            )

        def fire_gather(s, p):
            t = s // SLOTS_PER_TB
            lc = s % SLOTS_PER_TB
            for li in range(LC):
                pltpu.make_async_copy(
                    table_hbm.at[idxt_v.at[t, lc * LC + li]],
                    gbuf[p].at[pl.ds(li * 128, 128)],
                    gsem[p],
                ).start()

        def wait_gather(p):
            for li in range(LC):
                pltpu.make_async_copy(
                    table_hbm.at[idxt_v.at[0, li]],
                    gbuf[p].at[pl.ds(li * 128, 128)],
                    gsem[p],
                ).wait()

        def relu_chunk(p):
            src = gbuf[p]

            def body(r, c):
                for li in range(LC):
                    row = li * 128 + r
                    a0 = src[row, pl.ds(0, 16)]
                    a1 = src[row, pl.ds(16, 16)]
                    src[row, pl.ds(0, 16)] = jnp.maximum(a0, 0.0)
                    src[row, pl.ds(16, 16)] = jnp.maximum(a1, 0.0)
                return c

            lax.fori_loop(0, 128, body, 0)

        def fire_out(s, p):
            t = s // SLOTS_PER_TB
            lc = s % SLOTS_PER_TB
            src = gbuf[p]

            def body(u, c):
                li = u // NTD
                td = u % NTD
                for r in range(8):
                    pltpu.make_async_copy(
                        src.at[pl.ds(li * 128, 128), pl.ds(td * 8 + r, 1)],
                        out_hbm.at[lc * LC + li, td, tb0 + t, r],
                        osem[p],
                    ).start()
                return c

            lax.fori_loop(0, LC * NTD, body, 0)

        def drain_out(p):
            src = gbuf[p]

            def body(u, c):
                for r in range(8):
                    pltpu.make_async_copy(
                        src.at[pl.ds(0, 128), pl.ds(r, 1)],
                        out_hbm.at[0, 0, 0, r],
                        osem[p],
                    ).wait()
                return c

            lax.fori_loop(0, LC * NTD, body, 0)

        fire_gather(0, 0)
        fire_gather(1, 1)

        def loop_body(i, carry):
            s0 = i * 4
            for q in range(4):
                s = s0 + q
                wait_gather(q)
                relu_chunk(q)
                fire_out(s, q)

                @pl.when(s >= 2)
                def _(q=q):
                    drain_out((q + 2) % 4)

                @pl.when(s + 2 < NSLOTS)
                def _(s=s, q=q):
                    fire_gather(s + 2, (q + 2) % 4)

            return carry

        lax.fori_loop(0, NSLOTS // 4, loop_body, 0)

        drain_out(2)
        drain_out(3)

    return k(idx_t, table)


def kernel(predicate_indices, embed_weight):
    idx_t = jnp.swapaxes(predicate_indices.astype(jnp.int32), 0, 1)
    out6 = _sc_gather_relu(idx_t, embed_weight)
    out5 = out6.reshape(L, NTD, NTB, 8, 128)
    return out5.transpose(2, 4, 0, 1, 3).reshape(B, L, EMBED_DIM)


# speedup vs baseline: 53.2913x; 53.2913x over previous
import functools

import jax
import jax.numpy as jnp
from jax import lax
from jax.experimental import pallas as pl
from jax.experimental.pallas import tpu as pltpu
from jax.experimental.pallas import tpu_sc as plsc

VOCAB = 1000000
EMBED_DIM = 32
B = 16384
L = 50

NUM_WORKERS = 32
ROWS_PER_WORKER = B // NUM_WORKERS

CHUNK_ROWS = 8
NCHUNKS = ROWS_PER_WORKER // CHUNK_ROWS


def _sc_gather_relu(idx, table):
    mesh = plsc.VectorSubcoreMesh(core_axis_name="c", subcore_axis_name="s")

    @functools.partial(
        pl.kernel,
        mesh=mesh,
        out_type=jax.ShapeDtypeStruct((B * L * EMBED_DIM // 128, 128), jnp.float32),
        scratch_types=[
            pltpu.VMEM((ROWS_PER_WORKER, L), jnp.int32),
            pltpu.VMEM((CHUNK_ROWS, L, EMBED_DIM), jnp.float32),
            pltpu.VMEM((CHUNK_ROWS, L, EMBED_DIM), jnp.float32),
            pltpu.VMEM((CHUNK_ROWS * L * EMBED_DIM // 128, 128), jnp.float32),
            pltpu.VMEM((CHUNK_ROWS * L * EMBED_DIM // 128, 128), jnp.float32),
            pltpu.SemaphoreType.DMA,
            pltpu.SemaphoreType.DMA,
            pltpu.SemaphoreType.DMA,
            pltpu.SemaphoreType.DMA,
        ],
        compiler_params=pltpu.CompilerParams(use_tc_tiling_on_sc=False),
    )
    def k(idx_hbm, table_hbm, out_hbm, idx_v, g0, g1, o0, o1,
          gs0, gs1, os0, os1):
        wid = lax.axis_index("c") * 16 + lax.axis_index("s")
        row_base = wid * ROWS_PER_WORKER

        gbuf = (g0, g1)
        obuf = (o0, o1)
        gsem = (gs0, gs1)
        osem = (os0, os1)

        pltpu.sync_copy(idx_hbm.at[pl.ds(row_base, ROWS_PER_WORKER)], idx_v)

        def fire_gather(g, b):
            for j in range(CHUNK_ROWS):
                pltpu.make_async_copy(
                    table_hbm.at[idx_v.at[g * CHUNK_ROWS + j]],
                    gbuf[b].at[j],
                    gsem[b],
                ).start()

        def wait_gather(b):
            for j in range(CHUNK_ROWS):
                pltpu.make_async_copy(
                    table_hbm.at[idx_v.at[j]],
                    gbuf[b].at[j],
                    gsem[b],
                ).wait()

        def relu_chunk(b):
            src = gbuf[b]
            dst = obuf[b]

            def body(r2, carry):
                base = r2 * 128
                for k in range(8):
                    off = base + 16 * k
                    j = off // (L * EMBED_DIM)
                    rem = off % (L * EMBED_DIM)
                    r = rem // EMBED_DIM
                    c = rem % EMBED_DIM
                    a = src[j, r, pl.ds(c, 16)]
                    dst[r2, pl.ds(16 * k, 16)] = jnp.maximum(a, 0.0)
                return carry

            lax.fori_loop(0, CHUNK_ROWS * L * EMBED_DIM // 128, body, 0)

        OUT_CHUNK = CHUNK_ROWS * L * EMBED_DIM // 128

        def fire_out(g, b):
            pltpu.make_async_copy(
                obuf[b],
                out_hbm.at[pl.ds((row_base + g * CHUNK_ROWS) * L * EMBED_DIM // 128,
                                 OUT_CHUNK)],
                osem[b],
            ).start()

        def drain_out(b):
            pltpu.make_async_copy(
                obuf[b],
                out_hbm.at[pl.ds(0, OUT_CHUNK)],
                osem[b],
            ).wait()

        fire_gather(0, 0)
        fire_gather(1, 1)

        def loop_body(i, carry):
            g = i * 2
            for b in (0, 1):
                wait_gather(b)

                @pl.when(g + b >= 2)
                def _():
                    drain_out(b)

                relu_chunk(b)
                fire_out(g + b, b)

                @pl.when(g + b + 2 < NCHUNKS)
                def _(b=b):
                    fire_gather(g + b + 2, b)

            return carry

        lax.fori_loop(0, NCHUNKS // 2, loop_body, 0)

        drain_out(0)
        drain_out(1)

    return k(idx, table)


def kernel(predicate_indices, embed_weight):
    out2 = _sc_gather_relu(predicate_indices.astype(jnp.int32), embed_weight)
    return out2.reshape(B, L, EMBED_DIM)


# speedup vs baseline: 60.1676x; 1.1290x over previous
import functools

import jax
import jax.numpy as jnp
from jax import lax
from jax.experimental import pallas as pl
from jax.experimental.pallas import tpu as pltpu
from jax.experimental.pallas import tpu_sc as plsc

VOCAB = 1000000
EMBED_DIM = 32
B = 16384
L = 50

NUM_WORKERS = 32
ROWS_PER_WORKER = B // NUM_WORKERS

CHUNK_ROWS = 8
NCHUNKS = ROWS_PER_WORKER // CHUNK_ROWS


def _sc_gather_relu(idx, table):
    mesh = plsc.VectorSubcoreMesh(core_axis_name="c", subcore_axis_name="s")

    @functools.partial(
        pl.kernel,
        mesh=mesh,
        out_type=jax.ShapeDtypeStruct((B, L, EMBED_DIM), jnp.float32),
        scratch_types=[
            pltpu.VMEM((ROWS_PER_WORKER, L), jnp.int32),
            pltpu.VMEM((CHUNK_ROWS, L, EMBED_DIM), jnp.float32),
            pltpu.VMEM((CHUNK_ROWS, L, EMBED_DIM), jnp.float32),
            pltpu.VMEM((CHUNK_ROWS, L, EMBED_DIM), jnp.float32),
            pltpu.VMEM((CHUNK_ROWS, L, EMBED_DIM), jnp.float32),
            pltpu.SemaphoreType.DMA,
            pltpu.SemaphoreType.DMA,
            pltpu.SemaphoreType.DMA,
            pltpu.SemaphoreType.DMA,
        ],
        compiler_params=pltpu.CompilerParams(use_tc_tiling_on_sc=False),
    )
    def k(idx_hbm, table_hbm, out_hbm, idx_v, g0, g1, o0, o1,
          gs0, gs1, os0, os1):
        wid = lax.axis_index("c") * 16 + lax.axis_index("s")
        row_base = wid * ROWS_PER_WORKER

        gbuf = (g0, g1)
        obuf = (o0, o1)
        gsem = (gs0, gs1)
        osem = (os0, os1)

        pltpu.sync_copy(idx_hbm.at[pl.ds(row_base, ROWS_PER_WORKER)], idx_v)

        def fire_gather(g, b):
            for j in range(CHUNK_ROWS):
                pltpu.make_async_copy(
                    table_hbm.at[idx_v.at[g * CHUNK_ROWS + j]],
                    gbuf[b].at[j],
                    gsem[b],
                ).start()

        def wait_gather(b):
            for j in range(CHUNK_ROWS):
                pltpu.make_async_copy(
                    table_hbm.at[idx_v.at[j]],
                    gbuf[b].at[j],
                    gsem[b],
                ).wait()

        def relu_chunk(b):
            src = gbuf[b]
            dst = obuf[b]

            def body(r, c):
                for j in range(CHUNK_ROWS):
                    a0 = src[j, r, pl.ds(0, 16)]
                    a1 = src[j, r, pl.ds(16, 16)]
                    dst[j, r, pl.ds(0, 16)] = jnp.maximum(a0, 0.0)
                    dst[j, r, pl.ds(16, 16)] = jnp.maximum(a1, 0.0)
                return c

            lax.fori_loop(0, L, body, 0)

        def fire_out(g, b):
            pltpu.make_async_copy(
                obuf[b],
                out_hbm.at[pl.ds(row_base + g * CHUNK_ROWS, CHUNK_ROWS)],
                osem[b],
            ).start()

        def drain_out(b):
            pltpu.make_async_copy(
                obuf[b],
                out_hbm.at[pl.ds(row_base, CHUNK_ROWS)],
                osem[b],
            ).wait()

        fire_gather(0, 0)
        fire_gather(1, 1)

        def loop_body(i, carry):
            g = i * 2
            for b in (0, 1):
                wait_gather(b)

                @pl.when(g + b >= 2)
                def _():
                    drain_out(b)

                relu_chunk(b)
                fire_out(g + b, b)

                @pl.when(g + b + 2 < NCHUNKS)
                def _(b=b):
                    fire_gather(g + b + 2, b)

            return carry

        lax.fori_loop(0, NCHUNKS // 2, loop_body, 0)

        drain_out(0)
        drain_out(1)

    return k(idx, table)


def kernel(predicate_indices, embed_weight):
    return _sc_gather_relu(predicate_indices.astype(jnp.int32), embed_weight)
